# baseline (speedup 1.0000x reference)
import functools

import jax
import jax.numpy as jnp
from jax import lax
from jax.experimental import pallas as pl
from jax.experimental.pallas import tpu as pltpu
from jax.experimental.pallas import tpu_sc as plsc

N = 10000
E = 320000
F = 128
G = 64

NC = 2
NS = 16
CHUNK = 80
CHUNKS_PER_W = 125
NPAD = 10240
ROWS_PER_SUB = NPAD // NS
ZROWS = 16

_HIGH = jax.lax.Precision.HIGHEST


CPW = CHUNKS_PER_W
NBUF = 2
DRING = 2 * NBUF
ZROWS2 = 8


def _edge_agg(u, src3, dst3):
    mesh = plsc.VectorSubcoreMesh(core_axis_name="c", subcore_axis_name="s")

    @functools.partial(
        pl.kernel,
        out_type=jax.ShapeDtypeStruct((NC, NPAD, F), jnp.float32),
        mesh=mesh,
        scratch_types=[
            pltpu.VMEM((NBUF, CHUNK), jnp.int32),
            pltpu.VMEM((DRING, CHUNK), jnp.int32),
            pltpu.VMEM((NBUF, CHUNK, F), jnp.float32),
            pltpu.VMEM((ZROWS2, F), jnp.float32),
            pltpu.VMEM_SHARED((NPAD, F), jnp.float32),
            pltpu.SemaphoreType.DMA,
        ] + [pltpu.SemaphoreType.DMA] * (3 * NBUF),
    )
    def k(u_hbm, src_hbm, dst_hbm, agg_hbm,
          src_v, dst_v, rows_v, zf_v, acc_sh, zsem, *sems):
        isem = sems[:NBUF]
        gsem = sems[NBUF:2 * NBUF]
        ssem = sems[2 * NBUF:]
        cid = lax.axis_index("c")
        sid = lax.axis_index("s")
        wid = cid * NS + sid

        for b in range(NBUF):
            pltpu.async_copy(src_hbm.at[wid, b], src_v.at[b], isem[b])
            pltpu.async_copy(dst_hbm.at[wid, b], dst_v.at[b], isem[b])

        @pl.loop(0, ZROWS2)
        def _(i):
            @pl.loop(0, F // 16)
            def _(j):
                zf_v[i, pl.ds(j * 16, 16)] = jnp.zeros((16,), jnp.float32)

        zcopies = [
            pltpu.async_copy(
                zf_v,
                acc_sh.at[pl.ds(sid * ROWS_PER_SUB + z * ZROWS2, ZROWS2)],
                zsem)
            for z in range(ROWS_PER_SUB // ZROWS2)
        ]
        for c in zcopies:
            c.wait()

        plsc.subcore_barrier()

        NITER = (CPW + 2 * NBUF - 1) // (2 * NBUF)

        @pl.loop(0, NITER)
        def _(ii):
            base = ii * 2 * NBUF
            for r in range(2):
                gathers = [None] * NBUF
                for b in range(NBUF):
                    j = base + r * NBUF + b

                    @pl.when(j < CPW)
                    def _(b=b, j=j):
                        pltpu.make_async_copy(
                            src_hbm.at[wid, 0], src_v.at[b], isem[b]).wait()
                        pltpu.make_async_copy(
                            dst_hbm.at[wid, 0], dst_v.at[b % DRING],
                            isem[b]).wait()

                    @pl.when(jnp.logical_and(j < CPW, j >= NBUF))
                    def _(b=b):
                        pltpu.make_async_copy(
                            rows_v.at[b], acc_sh.at[dst_v.at[b]],
                            ssem[b]).wait()

                    @pl.when(j < CPW)
                    def _(b=b):
                        gathers[b] = pltpu.async_copy(
                            u_hbm.at[src_v.at[b]], rows_v.at[b], gsem[b])
                for b in range(NBUF):
                    j = base + r * NBUF + b
                    ds = r * NBUF + b
                    dsn = ((r + 1) % 2) * NBUF + b

                    @pl.when(j < CPW)
                    def _(b=b, ds=ds):
                        pltpu.make_async_copy(
                            u_hbm.at[src_v.at[b]], rows_v.at[b],
                            gsem[b]).wait()
                        pltpu.async_copy(rows_v.at[b], acc_sh.at[dst_v.at[ds]],
                                         ssem[b], add=True)

                    @pl.when(j + NBUF < CPW)
                    def _(b=b, j=j, dsn=dsn):
                        jn = j + NBUF
                        pltpu.async_copy(src_hbm.at[wid, jn], src_v.at[b],
                                         isem[b])
                        pltpu.async_copy(dst_hbm.at[wid, jn], dst_v.at[dsn],
                                        isem[b])

        for b in range(NBUF):
            pltpu.make_async_copy(
                rows_v.at[b], acc_sh.at[dst_v.at[b]], ssem[b]).wait()

        plsc.subcore_barrier()

        rr = sid * ROWS_PER_SUB
        pltpu.sync_copy(acc_sh.at[pl.ds(rr, ROWS_PER_SUB)],
                        agg_hbm.at[cid, pl.ds(rr, ROWS_PER_SUB)])

    return k(u, src3, dst3)


_BLK = 2000


def _mm2_body(x_ref, wl_ref, wr_ref, u_ref, t_ref):
    xb = x_ref[...]
    u_ref[...] = jnp.dot(xb, wl_ref[...], preferred_element_type=jnp.float32,
                         precision=_HIGH)
    t_ref[...] = jnp.dot(xb, wr_ref[...], preferred_element_type=jnp.float32,
                         precision=_HIGH)


def _mm2(x, wlT, wrT):
    return pl.pallas_call(
        _mm2_body,
        grid=(N // _BLK,),
        in_specs=[
            pl.BlockSpec((_BLK, F), lambda i: (i, 0)),
            pl.BlockSpec((F, F), lambda i: (0, 0)),
            pl.BlockSpec((F, F), lambda i: (0, 0)),
        ],
        out_specs=[
            pl.BlockSpec((_BLK, F), lambda i: (i, 0)),
            pl.BlockSpec((_BLK, F), lambda i: (i, 0)),
        ],
        out_shape=[
            jax.ShapeDtypeStruct((N, F), jnp.float32),
            jax.ShapeDtypeStruct((N, F), jnp.float32),
        ],
    )(x, wlT, wrT)


HI = NPAD // F
_HBLK = 4000


def _hist_body(dst_ref, cnt_ref):
    i = pl.program_id(0)
    d = dst_ref[...]
    hi = lax.shift_right_logical(d, 7)
    lo = jnp.bitwise_and(d, 127)
    ohh = (hi == lax.broadcasted_iota(jnp.int32, (_HBLK, HI), 1)
           ).astype(jnp.float32)
    ohl = (lo == lax.broadcasted_iota(jnp.int32, (_HBLK, F), 1)
           ).astype(jnp.float32)
    c = lax.dot_general(ohh, ohl, (((0,), (0,)), ((), ())),
                        preferred_element_type=jnp.float32)

    @pl.when(i == 0)
    def _():
        cnt_ref[...] = c

    @pl.when(i > 0)
    def _():
        cnt_ref[...] += c


def _hist(dst2d):
    return pl.pallas_call(
        _hist_body,
        grid=(E // _HBLK,),
        in_specs=[pl.BlockSpec((_HBLK, 1), lambda i: (i, 0))],
        out_specs=pl.BlockSpec((HI, F), lambda i: (0, 0)),
        out_shape=jax.ShapeDtypeStruct((HI, F), jnp.float32),
    )(dst2d)


def _combine_body(agg_ref, cnt_ref, t_ref, b_ref, wl_ref, wr_ref,
                  u_ref, t2_ref):
    agg = agg_ref[0] + agg_ref[1]
    inv = 1.0 / jnp.maximum(cnt_ref[...], 1.0)
    h = jnp.maximum(agg * inv + t_ref[...] + b_ref[...], 0.0)
    u_ref[...] = jnp.dot(h, wl_ref[...], preferred_element_type=jnp.float32,
                         precision=_HIGH)
    t2_ref[...] = jnp.dot(h, wr_ref[...], preferred_element_type=jnp.float32,
                          precision=_HIGH)


def _combine(agg, cnt, t, b, wlT, wrT):
    return pl.pallas_call(
        _combine_body,
        grid=(N // _BLK,),
        in_specs=[
            pl.BlockSpec((NC, _BLK, F), lambda i: (0, i, 0)),
            pl.BlockSpec((_BLK, 1), lambda i: (i, 0)),
            pl.BlockSpec((_BLK, F), lambda i: (i, 0)),
            pl.BlockSpec((1, F), lambda i: (0, 0)),
            pl.BlockSpec((F, F), lambda i: (0, 0)),
            pl.BlockSpec((F, F), lambda i: (0, 0)),
        ],
        out_specs=[
            pl.BlockSpec((_BLK, F), lambda i: (i, 0)),
            pl.BlockSpec((_BLK, F), lambda i: (i, 0)),
        ],
        out_shape=[
            jax.ShapeDtypeStruct((N, F), jnp.float32),
            jax.ShapeDtypeStruct((N, F), jnp.float32),
        ],
    )(agg, cnt, t, b, wlT, wrT)


def _pool_body(agg_ref, cnt_ref, t_ref, b_ref, batch_ref,
               wp_ref, bp_ref, wg_ref, bg_ref, gv_ref, gate_ref, pe_ref):
    i = pl.program_id(0)
    agg = agg_ref[0] + agg_ref[1]
    inv = 1.0 / jnp.maximum(cnt_ref[...], 1.0)
    h = jnp.maximum(agg * inv + t_ref[...] + b_ref[...], 0.0)
    he = jnp.concatenate([h, jnp.ones_like(h)], axis=1)
    oh = (batch_ref[...] ==
          lax.broadcasted_iota(jnp.int32, (_BLK, G), 1)).astype(jnp.float32)
    pe = lax.dot_general(oh, he, (((0,), (0,)), ((), ())),
                         preferred_element_type=jnp.float32, precision=_HIGH)

    @pl.when(i == 0)
    def _():
        pe_ref[...] = pe

    @pl.when(i > 0)
    def _():
        pe_ref[...] += pe

    @pl.when(i == N // _BLK - 1)
    def _():
        s = pe_ref[:, :F]
        c = pe_ref[:, F:F + 1]
        pooled = s * (1.0 / jnp.maximum(c, 1.0))
        gv = jnp.dot(pooled, wp_ref[...], preferred_element_type=jnp.float32,
                     precision=_HIGH) + bp_ref[...]
        gv_ref[...] = gv
        gate_ref[...] = jnp.dot(gv, wg_ref[...],
                                preferred_element_type=jnp.float32,
                                precision=_HIGH) + bg_ref[...]


def _pool(agg, cnt, t, b, batch2d, wpT, bp, wgT, bg):
    return pl.pallas_call(
        _pool_body,
        grid=(N // _BLK,),
        in_specs=[
            pl.BlockSpec((NC, _BLK, F), lambda i: (0, i, 0)),
            pl.BlockSpec((_BLK, 1), lambda i: (i, 0)),
            pl.BlockSpec((_BLK, F), lambda i: (i, 0)),
            pl.BlockSpec((1, F), lambda i: (0, 0)),
            pl.BlockSpec((_BLK, 1), lambda i: (i, 0)),
            pl.BlockSpec((F, F), lambda i: (0, 0)),
            pl.BlockSpec((1, F), lambda i: (0, 0)),
            pl.BlockSpec((F, F), lambda i: (0, 0)),
            pl.BlockSpec((1, F), lambda i: (0, 0)),
        ],
        out_specs=[
            pl.BlockSpec((G, F), lambda i: (0, 0)),
            pl.BlockSpec((G, F), lambda i: (0, 0)),
            pl.BlockSpec((G, 2 * F), lambda i: (0, 0)),
        ],
        out_shape=[
            jax.ShapeDtypeStruct((G, F), jnp.float32),
            jax.ShapeDtypeStruct((G, F), jnp.float32),
            jax.ShapeDtypeStruct((G, 2 * F), jnp.float32),
        ],
    )(agg, cnt, t, b, batch2d, wpT, bp, wgT, bg)


def kernel(x, W1l, b1l, W1r, W2l, b2l, W2r, Wp, bp, Wg, bg, edge_index, batch):
    w1lT, w1rT = W1l.T, W1r.T
    w2lT, w2rT = W2l.T, W2r.T
    wpT = Wp.T
    wgT = (Wg[:, :F] + Wg[:, F:]).T
    b1 = b1l.reshape(1, F)
    b2 = b2l.reshape(1, F)
    bp2 = bp.reshape(1, F)
    bg2 = bg.reshape(1, F)
    batch2d = batch.reshape(N, 1)
    src3 = edge_index[0].reshape(NC * NS, CPW, CHUNK)
    dst3 = edge_index[1].reshape(NC * NS, CPW, CHUNK)

    cnt = _hist(edge_index[1].reshape(E, 1)).reshape(NPAD, 1)
    u1, t1 = _mm2(x, w1lT, w1rT)
    agg1 = _edge_agg(u1, src3, dst3)
    u2, t2 = _combine(agg1, cnt, t1, b1, w2lT, w2rT)
    agg2 = _edge_agg(u2, src3, dst3)
    gv, gate, _ = _pool(agg2, cnt, t2, b2, batch2d, wpT, bp2, wgT, bg2)
    return gv, gate


# speedup vs baseline: 1.0394x; 1.0394x over previous
import functools

import jax
import jax.numpy as jnp
from jax import lax
from jax.experimental import pallas as pl
from jax.experimental.pallas import tpu as pltpu
from jax.experimental.pallas import tpu_sc as plsc

N = 10000
E = 320000
F = 128
G = 64

NC = 2
NS = 16
CHUNK = 40
CHUNKS_PER_W = 250
NPAD = 10240
ROWS_PER_SUB = NPAD // NS
ZROWS = 16

_HIGH = jax.lax.Precision.HIGHEST


CPW = CHUNKS_PER_W
NBUF = 4
DRING = 2 * NBUF
ZROWS2 = 8


def _edge_agg(u, src3, dst3):
    mesh = plsc.VectorSubcoreMesh(core_axis_name="c", subcore_axis_name="s")

    @functools.partial(
        pl.kernel,
        out_type=jax.ShapeDtypeStruct((NC, NPAD, F), jnp.float32),
        mesh=mesh,
        scratch_types=[
            pltpu.VMEM((NBUF, CHUNK), jnp.int32),
            pltpu.VMEM((DRING, CHUNK), jnp.int32),
            pltpu.VMEM((NBUF, CHUNK, F), jnp.float32),
            pltpu.VMEM((ZROWS2, F), jnp.float32),
            pltpu.VMEM_SHARED((NPAD, F), jnp.float32),
            pltpu.SemaphoreType.DMA,
        ] + [pltpu.SemaphoreType.DMA] * (3 * NBUF),
    )
    def k(u_hbm, src_hbm, dst_hbm, agg_hbm,
          src_v, dst_v, rows_v, zf_v, acc_sh, zsem, *sems):
        isem = sems[:NBUF]
        gsem = sems[NBUF:2 * NBUF]
        ssem = sems[2 * NBUF:]
        cid = lax.axis_index("c")
        sid = lax.axis_index("s")
        wid = cid * NS + sid

        for b in range(NBUF):
            pltpu.async_copy(src_hbm.at[wid, b], src_v.at[b], isem[b])
            pltpu.async_copy(dst_hbm.at[wid, b], dst_v.at[b], isem[b])

        @pl.loop(0, ZROWS2)
        def _(i):
            @pl.loop(0, F // 16)
            def _(j):
                zf_v[i, pl.ds(j * 16, 16)] = jnp.zeros((16,), jnp.float32)

        zcopies = [
            pltpu.async_copy(
                zf_v,
                acc_sh.at[pl.ds(sid * ROWS_PER_SUB + z * ZROWS2, ZROWS2)],
                zsem)
            for z in range(ROWS_PER_SUB // ZROWS2)
        ]
        for c in zcopies:
            c.wait()

        plsc.subcore_barrier()

        NITER = (CPW + 2 * NBUF - 1) // (2 * NBUF)

        @pl.loop(0, NITER)
        def _(ii):
            base = ii * 2 * NBUF
            for r in range(2):
                gathers = [None] * NBUF
                for b in range(NBUF):
                    j = base + r * NBUF + b

                    @pl.when(j < CPW)
                    def _(b=b, j=j):
                        pltpu.make_async_copy(
                            src_hbm.at[wid, 0], src_v.at[b], isem[b]).wait()
                        pltpu.make_async_copy(
                            dst_hbm.at[wid, 0], dst_v.at[b % DRING],
                            isem[b]).wait()

                    @pl.when(jnp.logical_and(j < CPW, j >= NBUF))
                    def _(b=b):
                        pltpu.make_async_copy(
                            rows_v.at[b], acc_sh.at[dst_v.at[b]],
                            ssem[b]).wait()

                    @pl.when(j < CPW)
                    def _(b=b):
                        gathers[b] = pltpu.async_copy(
                            u_hbm.at[src_v.at[b]], rows_v.at[b], gsem[b])
                for b in range(NBUF):
                    j = base + r * NBUF + b
                    ds = r * NBUF + b
                    dsn = ((r + 1) % 2) * NBUF + b

                    @pl.when(j < CPW)
                    def _(b=b, ds=ds):
                        pltpu.make_async_copy(
                            u_hbm.at[src_v.at[b]], rows_v.at[b],
                            gsem[b]).wait()
                        pltpu.async_copy(rows_v.at[b], acc_sh.at[dst_v.at[ds]],
                                         ssem[b], add=True)

                    @pl.when(j + NBUF < CPW)
                    def _(b=b, j=j, dsn=dsn):
                        jn = j + NBUF
                        pltpu.async_copy(src_hbm.at[wid, jn], src_v.at[b],
                                         isem[b])
                        pltpu.async_copy(dst_hbm.at[wid, jn], dst_v.at[dsn],
                                        isem[b])

        for b in range(NBUF):
            pltpu.make_async_copy(
                rows_v.at[b], acc_sh.at[dst_v.at[b]], ssem[b]).wait()

        plsc.subcore_barrier()

        rr = sid * ROWS_PER_SUB
        pltpu.sync_copy(acc_sh.at[pl.ds(rr, ROWS_PER_SUB)],
                        agg_hbm.at[cid, pl.ds(rr, ROWS_PER_SUB)])

    return k(u, src3, dst3)


_BLK = 2000


def _mm2_body(x_ref, wl_ref, wr_ref, u_ref, t_ref):
    xb = x_ref[...]
    u_ref[...] = jnp.dot(xb, wl_ref[...], preferred_element_type=jnp.float32,
                         precision=_HIGH)
    t_ref[...] = jnp.dot(xb, wr_ref[...], preferred_element_type=jnp.float32,
                         precision=_HIGH)


def _mm2(x, wlT, wrT):
    return pl.pallas_call(
        _mm2_body,
        grid=(N // _BLK,),
        in_specs=[
            pl.BlockSpec((_BLK, F), lambda i: (i, 0)),
            pl.BlockSpec((F, F), lambda i: (0, 0)),
            pl.BlockSpec((F, F), lambda i: (0, 0)),
        ],
        out_specs=[
            pl.BlockSpec((_BLK, F), lambda i: (i, 0)),
            pl.BlockSpec((_BLK, F), lambda i: (i, 0)),
        ],
        out_shape=[
            jax.ShapeDtypeStruct((N, F), jnp.float32),
            jax.ShapeDtypeStruct((N, F), jnp.float32),
        ],
    )(x, wlT, wrT)


HI = NPAD // F
_HBLK = 4000


def _hist_body(dst_ref, cnt_ref):
    i = pl.program_id(0)
    d = dst_ref[...]
    hi = lax.shift_right_logical(d, 7)
    lo = jnp.bitwise_and(d, 127)
    ohh = (hi == lax.broadcasted_iota(jnp.int32, (_HBLK, HI), 1)
           ).astype(jnp.float32)
    ohl = (lo == lax.broadcasted_iota(jnp.int32, (_HBLK, F), 1)
           ).astype(jnp.float32)
    c = lax.dot_general(ohh, ohl, (((0,), (0,)), ((), ())),
                        preferred_element_type=jnp.float32)

    @pl.when(i == 0)
    def _():
        cnt_ref[...] = c

    @pl.when(i > 0)
    def _():
        cnt_ref[...] += c


def _hist(dst2d):
    return pl.pallas_call(
        _hist_body,
        grid=(E // _HBLK,),
        in_specs=[pl.BlockSpec((_HBLK, 1), lambda i: (i, 0))],
        out_specs=pl.BlockSpec((HI, F), lambda i: (0, 0)),
        out_shape=jax.ShapeDtypeStruct((HI, F), jnp.float32),
    )(dst2d)


def _combine_body(agg_ref, cnt_ref, t_ref, b_ref, wl_ref, wr_ref,
                  u_ref, t2_ref):
    agg = agg_ref[0] + agg_ref[1]
    inv = 1.0 / jnp.maximum(cnt_ref[...], 1.0)
    h = jnp.maximum(agg * inv + t_ref[...] + b_ref[...], 0.0)
    u_ref[...] = jnp.dot(h, wl_ref[...], preferred_element_type=jnp.float32,
                         precision=_HIGH)
    t2_ref[...] = jnp.dot(h, wr_ref[...], preferred_element_type=jnp.float32,
                          precision=_HIGH)


def _combine(agg, cnt, t, b, wlT, wrT):
    return pl.pallas_call(
        _combine_body,
        grid=(N // _BLK,),
        in_specs=[
            pl.BlockSpec((NC, _BLK, F), lambda i: (0, i, 0)),
            pl.BlockSpec((_BLK, 1), lambda i: (i, 0)),
            pl.BlockSpec((_BLK, F), lambda i: (i, 0)),
            pl.BlockSpec((1, F), lambda i: (0, 0)),
            pl.BlockSpec((F, F), lambda i: (0, 0)),
            pl.BlockSpec((F, F), lambda i: (0, 0)),
        ],
        out_specs=[
            pl.BlockSpec((_BLK, F), lambda i: (i, 0)),
            pl.BlockSpec((_BLK, F), lambda i: (i, 0)),
        ],
        out_shape=[
            jax.ShapeDtypeStruct((N, F), jnp.float32),
            jax.ShapeDtypeStruct((N, F), jnp.float32),
        ],
    )(agg, cnt, t, b, wlT, wrT)


def _pool_body(agg_ref, cnt_ref, t_ref, b_ref, batch_ref,
               wp_ref, bp_ref, wg_ref, bg_ref, gv_ref, gate_ref, pe_ref):
    i = pl.program_id(0)
    agg = agg_ref[0] + agg_ref[1]
    inv = 1.0 / jnp.maximum(cnt_ref[...], 1.0)
    h = jnp.maximum(agg * inv + t_ref[...] + b_ref[...], 0.0)
    he = jnp.concatenate([h, jnp.ones_like(h)], axis=1)
    oh = (batch_ref[...] ==
          lax.broadcasted_iota(jnp.int32, (_BLK, G), 1)).astype(jnp.float32)
    pe = lax.dot_general(oh, he, (((0,), (0,)), ((), ())),
                         preferred_element_type=jnp.float32, precision=_HIGH)

    @pl.when(i == 0)
    def _():
        pe_ref[...] = pe

    @pl.when(i > 0)
    def _():
        pe_ref[...] += pe

    @pl.when(i == N // _BLK - 1)
    def _():
        s = pe_ref[:, :F]
        c = pe_ref[:, F:F + 1]
        pooled = s * (1.0 / jnp.maximum(c, 1.0))
        gv = jnp.dot(pooled, wp_ref[...], preferred_element_type=jnp.float32,
                     precision=_HIGH) + bp_ref[...]
        gv_ref[...] = gv
        gate_ref[...] = jnp.dot(gv, wg_ref[...],
                                preferred_element_type=jnp.float32,
                                precision=_HIGH) + bg_ref[...]


def _pool(agg, cnt, t, b, batch2d, wpT, bp, wgT, bg):
    return pl.pallas_call(
        _pool_body,
        grid=(N // _BLK,),
        in_specs=[
            pl.BlockSpec((NC, _BLK, F), lambda i: (0, i, 0)),
            pl.BlockSpec((_BLK, 1), lambda i: (i, 0)),
            pl.BlockSpec((_BLK, F), lambda i: (i, 0)),
            pl.BlockSpec((1, F), lambda i: (0, 0)),
            pl.BlockSpec((_BLK, 1), lambda i: (i, 0)),
            pl.BlockSpec((F, F), lambda i: (0, 0)),
            pl.BlockSpec((1, F), lambda i: (0, 0)),
            pl.BlockSpec((F, F), lambda i: (0, 0)),
            pl.BlockSpec((1, F), lambda i: (0, 0)),
        ],
        out_specs=[
            pl.BlockSpec((G, F), lambda i: (0, 0)),
            pl.BlockSpec((G, F), lambda i: (0, 0)),
            pl.BlockSpec((G, 2 * F), lambda i: (0, 0)),
        ],
        out_shape=[
            jax.ShapeDtypeStruct((G, F), jnp.float32),
            jax.ShapeDtypeStruct((G, F), jnp.float32),
            jax.ShapeDtypeStruct((G, 2 * F), jnp.float32),
        ],
    )(agg, cnt, t, b, batch2d, wpT, bp, wgT, bg)


def kernel(x, W1l, b1l, W1r, W2l, b2l, W2r, Wp, bp, Wg, bg, edge_index, batch):
    w1lT, w1rT = W1l.T, W1r.T
    w2lT, w2rT = W2l.T, W2r.T
    wpT = Wp.T
    wgT = (Wg[:, :F] + Wg[:, F:]).T
    b1 = b1l.reshape(1, F)
    b2 = b2l.reshape(1, F)
    bp2 = bp.reshape(1, F)
    bg2 = bg.reshape(1, F)
    batch2d = batch.reshape(N, 1)
    src3 = edge_index[0].reshape(NC * NS, CPW, CHUNK)
    dst3 = edge_index[1].reshape(NC * NS, CPW, CHUNK)

    cnt = _hist(edge_index[1].reshape(E, 1)).reshape(NPAD, 1)
    u1, t1 = _mm2(x, w1lT, w1rT)
    agg1 = _edge_agg(u1, src3, dst3)
    u2, t2 = _combine(agg1, cnt, t1, b1, w2lT, w2rT)
    agg2 = _edge_agg(u2, src3, dst3)
    gv, gate, _ = _pool(agg2, cnt, t2, b2, batch2d, wpT, bp2, wgT, bg2)
    return gv, gate


# speedup vs baseline: 1.0541x; 1.0142x over previous
import functools

import jax
import jax.numpy as jnp
from jax import lax
from jax.experimental import pallas as pl
from jax.experimental.pallas import tpu as pltpu
from jax.experimental.pallas import tpu_sc as plsc

N = 10000
E = 320000
F = 128
G = 64

NC = 2
NS = 16
CHUNK = 40
CHUNKS_PER_W = 250
NPAD = 10240
ROWS_PER_SUB = NPAD // NS
ZROWS = 16

_HIGH = jax.lax.Precision.HIGHEST


CPW = CHUNKS_PER_W
NBUF = 5
DRING = 2 * NBUF
ZROWS2 = 8


def _edge_agg(u, src3, dst3):
    mesh = plsc.VectorSubcoreMesh(core_axis_name="c", subcore_axis_name="s")

    @functools.partial(
        pl.kernel,
        out_type=jax.ShapeDtypeStruct((NC, NPAD, F), jnp.float32),
        mesh=mesh,
        scratch_types=[
            pltpu.VMEM((NBUF, CHUNK), jnp.int32),
            pltpu.VMEM((DRING, CHUNK), jnp.int32),
            pltpu.VMEM((NBUF, CHUNK, F), jnp.float32),
            pltpu.VMEM((ZROWS2, F), jnp.float32),
            pltpu.VMEM_SHARED((NPAD, F), jnp.float32),
            pltpu.SemaphoreType.DMA,
        ] + [pltpu.SemaphoreType.DMA] * (3 * NBUF),
    )
    def k(u_hbm, src_hbm, dst_hbm, agg_hbm,
          src_v, dst_v, rows_v, zf_v, acc_sh, zsem, *sems):
        isem = sems[:NBUF]
        gsem = sems[NBUF:2 * NBUF]
        ssem = sems[2 * NBUF:]
        cid = lax.axis_index("c")
        sid = lax.axis_index("s")
        wid = cid * NS + sid

        for b in range(NBUF):
            pltpu.async_copy(src_hbm.at[wid, b], src_v.at[b], isem[b])
            pltpu.async_copy(dst_hbm.at[wid, b], dst_v.at[b], isem[b])

        @pl.loop(0, ZROWS2)
        def _(i):
            @pl.loop(0, F // 16)
            def _(j):
                zf_v[i, pl.ds(j * 16, 16)] = jnp.zeros((16,), jnp.float32)

        zcopies = [
            pltpu.async_copy(
                zf_v,
                acc_sh.at[pl.ds(sid * ROWS_PER_SUB + z * ZROWS2, ZROWS2)],
                zsem)
            for z in range(ROWS_PER_SUB // ZROWS2)
        ]
        for c in zcopies:
            c.wait()

        plsc.subcore_barrier()

        NITER = (CPW + 2 * NBUF - 1) // (2 * NBUF)

        @pl.loop(0, NITER)
        def _(ii):
            base = ii * 2 * NBUF
            for r in range(2):
                gathers = [None] * NBUF
                for b in range(NBUF):
                    j = base + r * NBUF + b

                    @pl.when(j < CPW)
                    def _(b=b, j=j):
                        pltpu.make_async_copy(
                            src_hbm.at[wid, 0], src_v.at[b], isem[b]).wait()
                        pltpu.make_async_copy(
                            dst_hbm.at[wid, 0], dst_v.at[b % DRING],
                            isem[b]).wait()

                    @pl.when(jnp.logical_and(j < CPW, j >= NBUF))
                    def _(b=b):
                        pltpu.make_async_copy(
                            rows_v.at[b], acc_sh.at[dst_v.at[b]],
                            ssem[b]).wait()

                    @pl.when(j < CPW)
                    def _(b=b):
                        gathers[b] = pltpu.async_copy(
                            u_hbm.at[src_v.at[b]], rows_v.at[b], gsem[b])
                for b in range(NBUF):
                    j = base + r * NBUF + b
                    ds = r * NBUF + b
                    dsn = ((r + 1) % 2) * NBUF + b

                    @pl.when(j < CPW)
                    def _(b=b, ds=ds):
                        pltpu.make_async_copy(
                            u_hbm.at[src_v.at[b]], rows_v.at[b],
                            gsem[b]).wait()
                        pltpu.async_copy(rows_v.at[b], acc_sh.at[dst_v.at[ds]],
                                         ssem[b], add=True)

                    @pl.when(j + NBUF < CPW)
                    def _(b=b, j=j, dsn=dsn):
                        jn = j + NBUF
                        pltpu.async_copy(src_hbm.at[wid, jn], src_v.at[b],
                                         isem[b])
                        pltpu.async_copy(dst_hbm.at[wid, jn], dst_v.at[dsn],
                                        isem[b])

        for b in range(NBUF):
            pltpu.make_async_copy(
                rows_v.at[b], acc_sh.at[dst_v.at[b]], ssem[b]).wait()

        plsc.subcore_barrier()

        rr = sid * ROWS_PER_SUB
        pltpu.sync_copy(acc_sh.at[pl.ds(rr, ROWS_PER_SUB)],
                        agg_hbm.at[cid, pl.ds(rr, ROWS_PER_SUB)])

    return k(u, src3, dst3)


_BLK = 2000


def _mm2_body(x_ref, wl_ref, wr_ref, u_ref, t_ref):
    xb = x_ref[...]
    u_ref[...] = jnp.dot(xb, wl_ref[...], preferred_element_type=jnp.float32,
                         precision=_HIGH)
    t_ref[...] = jnp.dot(xb, wr_ref[...], preferred_element_type=jnp.float32,
                         precision=_HIGH)


def _mm2(x, wlT, wrT):
    return pl.pallas_call(
        _mm2_body,
        grid=(N // _BLK,),
        in_specs=[
            pl.BlockSpec((_BLK, F), lambda i: (i, 0)),
            pl.BlockSpec((F, F), lambda i: (0, 0)),
            pl.BlockSpec((F, F), lambda i: (0, 0)),
        ],
        out_specs=[
            pl.BlockSpec((_BLK, F), lambda i: (i, 0)),
            pl.BlockSpec((_BLK, F), lambda i: (i, 0)),
        ],
        out_shape=[
            jax.ShapeDtypeStruct((N, F), jnp.float32),
            jax.ShapeDtypeStruct((N, F), jnp.float32),
        ],
    )(x, wlT, wrT)


HI = NPAD // F
_HBLK = 4000


def _hist_body(dst_ref, cnt_ref):
    i = pl.program_id(0)
    d = dst_ref[...]
    hi = lax.shift_right_logical(d, 7)
    lo = jnp.bitwise_and(d, 127)
    ohh = (hi == lax.broadcasted_iota(jnp.int32, (_HBLK, HI), 1)
           ).astype(jnp.float32)
    ohl = (lo == lax.broadcasted_iota(jnp.int32, (_HBLK, F), 1)
           ).astype(jnp.float32)
    c = lax.dot_general(ohh, ohl, (((0,), (0,)), ((), ())),
                        preferred_element_type=jnp.float32)

    @pl.when(i == 0)
    def _():
        cnt_ref[...] = c

    @pl.when(i > 0)
    def _():
        cnt_ref[...] += c


def _hist(dst2d):
    return pl.pallas_call(
        _hist_body,
        grid=(E // _HBLK,),
        in_specs=[pl.BlockSpec((_HBLK, 1), lambda i: (i, 0))],
        out_specs=pl.BlockSpec((HI, F), lambda i: (0, 0)),
        out_shape=jax.ShapeDtypeStruct((HI, F), jnp.float32),
    )(dst2d)


def _combine_body(agg_ref, cnt_ref, t_ref, b_ref, wl_ref, wr_ref,
                  u_ref, t2_ref):
    agg = agg_ref[0] + agg_ref[1]
    inv = 1.0 / jnp.maximum(cnt_ref[...], 1.0)
    h = jnp.maximum(agg * inv + t_ref[...] + b_ref[...], 0.0)
    u_ref[...] = jnp.dot(h, wl_ref[...], preferred_element_type=jnp.float32,
                         precision=_HIGH)
    t2_ref[...] = jnp.dot(h, wr_ref[...], preferred_element_type=jnp.float32,
                          precision=_HIGH)


def _combine(agg, cnt, t, b, wlT, wrT):
    return pl.pallas_call(
        _combine_body,
        grid=(N // _BLK,),
        in_specs=[
            pl.BlockSpec((NC, _BLK, F), lambda i: (0, i, 0)),
            pl.BlockSpec((_BLK, 1), lambda i: (i, 0)),
            pl.BlockSpec((_BLK, F), lambda i: (i, 0)),
            pl.BlockSpec((1, F), lambda i: (0, 0)),
            pl.BlockSpec((F, F), lambda i: (0, 0)),
            pl.BlockSpec((F, F), lambda i: (0, 0)),
        ],
        out_specs=[
            pl.BlockSpec((_BLK, F), lambda i: (i, 0)),
            pl.BlockSpec((_BLK, F), lambda i: (i, 0)),
        ],
        out_shape=[
            jax.ShapeDtypeStruct((N, F), jnp.float32),
            jax.ShapeDtypeStruct((N, F), jnp.float32),
        ],
    )(agg, cnt, t, b, wlT, wrT)


def _pool_body(agg_ref, cnt_ref, t_ref, b_ref, batch_ref,
               wp_ref, bp_ref, wg_ref, bg_ref, gv_ref, gate_ref, pe_ref):
    i = pl.program_id(0)
    agg = agg_ref[0] + agg_ref[1]
    inv = 1.0 / jnp.maximum(cnt_ref[...], 1.0)
    h = jnp.maximum(agg * inv + t_ref[...] + b_ref[...], 0.0)
    he = jnp.concatenate([h, jnp.ones_like(h)], axis=1)
    oh = (batch_ref[...] ==
          lax.broadcasted_iota(jnp.int32, (_BLK, G), 1)).astype(jnp.float32)
    pe = lax.dot_general(oh, he, (((0,), (0,)), ((), ())),
                         preferred_element_type=jnp.float32, precision=_HIGH)

    @pl.when(i == 0)
    def _():
        pe_ref[...] = pe

    @pl.when(i > 0)
    def _():
        pe_ref[...] += pe

    @pl.when(i == N // _BLK - 1)
    def _():
        s = pe_ref[:, :F]
        c = pe_ref[:, F:F + 1]
        pooled = s * (1.0 / jnp.maximum(c, 1.0))
        gv = jnp.dot(pooled, wp_ref[...], preferred_element_type=jnp.float32,
                     precision=_HIGH) + bp_ref[...]
        gv_ref[...] = gv
        gate_ref[...] = jnp.dot(gv, wg_ref[...],
                                preferred_element_type=jnp.float32,
                                precision=_HIGH) + bg_ref[...]


def _pool(agg, cnt, t, b, batch2d, wpT, bp, wgT, bg):
    return pl.pallas_call(
        _pool_body,
        grid=(N // _BLK,),
        in_specs=[
            pl.BlockSpec((NC, _BLK, F), lambda i: (0, i, 0)),
            pl.BlockSpec((_BLK, 1), lambda i: (i, 0)),
            pl.BlockSpec((_BLK, F), lambda i: (i, 0)),
            pl.BlockSpec((1, F), lambda i: (0, 0)),
            pl.BlockSpec((_BLK, 1), lambda i: (i, 0)),
            pl.BlockSpec((F, F), lambda i: (0, 0)),
            pl.BlockSpec((1, F), lambda i: (0, 0)),
            pl.BlockSpec((F, F), lambda i: (0, 0)),
            pl.BlockSpec((1, F), lambda i: (0, 0)),
        ],
        out_specs=[
            pl.BlockSpec((G, F), lambda i: (0, 0)),
            pl.BlockSpec((G, F), lambda i: (0, 0)),
            pl.BlockSpec((G, 2 * F), lambda i: (0, 0)),
        ],
        out_shape=[
            jax.ShapeDtypeStruct((G, F), jnp.float32),
            jax.ShapeDtypeStruct((G, F), jnp.float32),
            jax.ShapeDtypeStruct((G, 2 * F), jnp.float32),
        ],
    )(agg, cnt, t, b, batch2d, wpT, bp, wgT, bg)


def kernel(x, W1l, b1l, W1r, W2l, b2l, W2r, Wp, bp, Wg, bg, edge_index, batch):
    w1lT, w1rT = W1l.T, W1r.T
    w2lT, w2rT = W2l.T, W2r.T
    wpT = Wp.T
    wgT = (Wg[:, :F] + Wg[:, F:]).T
    b1 = b1l.reshape(1, F)
    b2 = b2l.reshape(1, F)
    bp2 = bp.reshape(1, F)
    bg2 = bg.reshape(1, F)
    batch2d = batch.reshape(N, 1)
    src3 = edge_index[0].reshape(NC * NS, CPW, CHUNK)
    dst3 = edge_index[1].reshape(NC * NS, CPW, CHUNK)

    cnt = _hist(edge_index[1].reshape(E, 1)).reshape(NPAD, 1)
    u1, t1 = _mm2(x, w1lT, w1rT)
    agg1 = _edge_agg(u1, src3, dst3)
    u2, t2 = _combine(agg1, cnt, t1, b1, w2lT, w2rT)
    agg2 = _edge_agg(u2, src3, dst3)
    gv, gate, _ = _pool(agg2, cnt, t2, b2, batch2d, wpT, bp2, wgT, bg2)
    return gv, gate


# speedup vs baseline: 1.0621x; 1.0076x over previous
import functools

import jax
import jax.numpy as jnp
from jax import lax
from jax.experimental import pallas as pl
from jax.experimental.pallas import tpu as pltpu
from jax.experimental.pallas import tpu_sc as plsc

N = 10000
E = 320000
F = 128
G = 64

NC = 2
NS = 16
CHUNK = 40
CHUNKS_PER_W = 250
NPAD = 10240
ROWS_PER_SUB = NPAD // NS
ZROWS = 16

_HIGH = jax.lax.Precision.HIGHEST


CPW = CHUNKS_PER_W
NBUF = 6
DRING = 2 * NBUF
ZROWS2 = 8


def _edge_agg(u, src3, dst3):
    mesh = plsc.VectorSubcoreMesh(core_axis_name="c", subcore_axis_name="s")

    @functools.partial(
        pl.kernel,
        out_type=jax.ShapeDtypeStruct((NC, NPAD, F), jnp.float32),
        mesh=mesh,
        scratch_types=[
            pltpu.VMEM((NBUF, CHUNK), jnp.int32),
            pltpu.VMEM((DRING, CHUNK), jnp.int32),
            pltpu.VMEM((NBUF, CHUNK, F), jnp.float32),
            pltpu.VMEM((ZROWS2, F), jnp.float32),
            pltpu.VMEM_SHARED((NPAD, F), jnp.float32),
            pltpu.SemaphoreType.DMA,
        ] + [pltpu.SemaphoreType.DMA] * (3 * NBUF),
    )
    def k(u_hbm, src_hbm, dst_hbm, agg_hbm,
          src_v, dst_v, rows_v, zf_v, acc_sh, zsem, *sems):
        isem = sems[:NBUF]
        gsem = sems[NBUF:2 * NBUF]
        ssem = sems[2 * NBUF:]
        cid = lax.axis_index("c")
        sid = lax.axis_index("s")
        wid = cid * NS + sid

        for b in range(NBUF):
            pltpu.async_copy(src_hbm.at[wid, b], src_v.at[b], isem[b])
            pltpu.async_copy(dst_hbm.at[wid, b], dst_v.at[b], isem[b])

        @pl.loop(0, ZROWS2)
        def _(i):
            @pl.loop(0, F // 16)
            def _(j):
                zf_v[i, pl.ds(j * 16, 16)] = jnp.zeros((16,), jnp.float32)

        zcopies = [
            pltpu.async_copy(
                zf_v,
                acc_sh.at[pl.ds(sid * ROWS_PER_SUB + z * ZROWS2, ZROWS2)],
                zsem)
            for z in range(ROWS_PER_SUB // ZROWS2)
        ]
        for c in zcopies:
            c.wait()

        plsc.subcore_barrier()

        NITER = (CPW + 2 * NBUF - 1) // (2 * NBUF)

        @pl.loop(0, NITER)
        def _(ii):
            base = ii * 2 * NBUF
            for r in range(2):
                gathers = [None] * NBUF
                for b in range(NBUF):
                    j = base + r * NBUF + b

                    @pl.when(j < CPW)
                    def _(b=b, j=j):
                        pltpu.make_async_copy(
                            src_hbm.at[wid, 0], src_v.at[b], isem[b]).wait()
                        pltpu.make_async_copy(
                            dst_hbm.at[wid, 0], dst_v.at[b % DRING],
                            isem[b]).wait()

                    @pl.when(jnp.logical_and(j < CPW, j >= NBUF))
                    def _(b=b):
                        pltpu.make_async_copy(
                            rows_v.at[b], acc_sh.at[dst_v.at[b]],
                            ssem[b]).wait()

                    @pl.when(j < CPW)
                    def _(b=b):
                        gathers[b] = pltpu.async_copy(
                            u_hbm.at[src_v.at[b]], rows_v.at[b], gsem[b])
                for b in range(NBUF):
                    j = base + r * NBUF + b
                    ds = r * NBUF + b
                    dsn = ((r + 1) % 2) * NBUF + b

                    @pl.when(j < CPW)
                    def _(b=b, ds=ds):
                        pltpu.make_async_copy(
                            u_hbm.at[src_v.at[b]], rows_v.at[b],
                            gsem[b]).wait()
                        pltpu.async_copy(rows_v.at[b], acc_sh.at[dst_v.at[ds]],
                                         ssem[b], add=True)

                    @pl.when(j + NBUF < CPW)
                    def _(b=b, j=j, dsn=dsn):
                        jn = j + NBUF
                        pltpu.async_copy(src_hbm.at[wid, jn], src_v.at[b],
                                         isem[b])
                        pltpu.async_copy(dst_hbm.at[wid, jn], dst_v.at[dsn],
                                        isem[b])

        for b in range(NBUF):
            pltpu.make_async_copy(
                rows_v.at[b], acc_sh.at[dst_v.at[b]], ssem[b]).wait()

        plsc.subcore_barrier()

        rr = sid * ROWS_PER_SUB
        pltpu.sync_copy(acc_sh.at[pl.ds(rr, ROWS_PER_SUB)],
                        agg_hbm.at[cid, pl.ds(rr, ROWS_PER_SUB)])

    return k(u, src3, dst3)


_BLK = 2000


def _mm2_body(x_ref, wl_ref, wr_ref, u_ref, t_ref):
    xb = x_ref[...]
    u_ref[...] = jnp.dot(xb, wl_ref[...], preferred_element_type=jnp.float32,
                         precision=_HIGH)
    t_ref[...] = jnp.dot(xb, wr_ref[...], preferred_element_type=jnp.float32,
                         precision=_HIGH)


def _mm2(x, wlT, wrT):
    return pl.pallas_call(
        _mm2_body,
        grid=(N // _BLK,),
        in_specs=[
            pl.BlockSpec((_BLK, F), lambda i: (i, 0)),
            pl.BlockSpec((F, F), lambda i: (0, 0)),
            pl.BlockSpec((F, F), lambda i: (0, 0)),
        ],
        out_specs=[
            pl.BlockSpec((_BLK, F), lambda i: (i, 0)),
            pl.BlockSpec((_BLK, F), lambda i: (i, 0)),
        ],
        out_shape=[
            jax.ShapeDtypeStruct((N, F), jnp.float32),
            jax.ShapeDtypeStruct((N, F), jnp.float32),
        ],
    )(x, wlT, wrT)


HI = NPAD // F
_HBLK = 4000


def _hist_body(dst_ref, cnt_ref):
    i = pl.program_id(0)
    d = dst_ref[...]
    hi = lax.shift_right_logical(d, 7)
    lo = jnp.bitwise_and(d, 127)
    ohh = (hi == lax.broadcasted_iota(jnp.int32, (_HBLK, HI), 1)
           ).astype(jnp.float32)
    ohl = (lo == lax.broadcasted_iota(jnp.int32, (_HBLK, F), 1)
           ).astype(jnp.float32)
    c = lax.dot_general(ohh, ohl, (((0,), (0,)), ((), ())),
                        preferred_element_type=jnp.float32)

    @pl.when(i == 0)
    def _():
        cnt_ref[...] = c

    @pl.when(i > 0)
    def _():
        cnt_ref[...] += c


def _hist(dst2d):
    return pl.pallas_call(
        _hist_body,
        grid=(E // _HBLK,),
        in_specs=[pl.BlockSpec((_HBLK, 1), lambda i: (i, 0))],
        out_specs=pl.BlockSpec((HI, F), lambda i: (0, 0)),
        out_shape=jax.ShapeDtypeStruct((HI, F), jnp.float32),
    )(dst2d)


def _combine_body(agg_ref, cnt_ref, t_ref, b_ref, wl_ref, wr_ref,
                  u_ref, t2_ref):
    agg = agg_ref[0] + agg_ref[1]
    inv = 1.0 / jnp.maximum(cnt_ref[...], 1.0)
    h = jnp.maximum(agg * inv + t_ref[...] + b_ref[...], 0.0)
    u_ref[...] = jnp.dot(h, wl_ref[...], preferred_element_type=jnp.float32,
                         precision=_HIGH)
    t2_ref[...] = jnp.dot(h, wr_ref[...], preferred_element_type=jnp.float32,
                          precision=_HIGH)


def _combine(agg, cnt, t, b, wlT, wrT):
    return pl.pallas_call(
        _combine_body,
        grid=(N // _BLK,),
        in_specs=[
            pl.BlockSpec((NC, _BLK, F), lambda i: (0, i, 0)),
            pl.BlockSpec((_BLK, 1), lambda i: (i, 0)),
            pl.BlockSpec((_BLK, F), lambda i: (i, 0)),
            pl.BlockSpec((1, F), lambda i: (0, 0)),
            pl.BlockSpec((F, F), lambda i: (0, 0)),
            pl.BlockSpec((F, F), lambda i: (0, 0)),
        ],
        out_specs=[
            pl.BlockSpec((_BLK, F), lambda i: (i, 0)),
            pl.BlockSpec((_BLK, F), lambda i: (i, 0)),
        ],
        out_shape=[
            jax.ShapeDtypeStruct((N, F), jnp.float32),
            jax.ShapeDtypeStruct((N, F), jnp.float32),
        ],
    )(agg, cnt, t, b, wlT, wrT)


def _pool_body(agg_ref, cnt_ref, t_ref, b_ref, batch_ref,
               wp_ref, bp_ref, wg_ref, bg_ref, gv_ref, gate_ref, pe_ref):
    i = pl.program_id(0)
    agg = agg_ref[0] + agg_ref[1]
    inv = 1.0 / jnp.maximum(cnt_ref[...], 1.0)
    h = jnp.maximum(agg * inv + t_ref[...] + b_ref[...], 0.0)
    he = jnp.concatenate([h, jnp.ones_like(h)], axis=1)
    oh = (batch_ref[...] ==
          lax.broadcasted_iota(jnp.int32, (_BLK, G), 1)).astype(jnp.float32)
    pe = lax.dot_general(oh, he, (((0,), (0,)), ((), ())),
                         preferred_element_type=jnp.float32, precision=_HIGH)

    @pl.when(i == 0)
    def _():
        pe_ref[...] = pe

    @pl.when(i > 0)
    def _():
        pe_ref[...] += pe

    @pl.when(i == N // _BLK - 1)
    def _():
        s = pe_ref[:, :F]
        c = pe_ref[:, F:F + 1]
        pooled = s * (1.0 / jnp.maximum(c, 1.0))
        gv = jnp.dot(pooled, wp_ref[...], preferred_element_type=jnp.float32,
                     precision=_HIGH) + bp_ref[...]
        gv_ref[...] = gv
        gate_ref[...] = jnp.dot(gv, wg_ref[...],
                                preferred_element_type=jnp.float32,
                                precision=_HIGH) + bg_ref[...]


def _pool(agg, cnt, t, b, batch2d, wpT, bp, wgT, bg):
    return pl.pallas_call(
        _pool_body,
        grid=(N // _BLK,),
        in_specs=[
            pl.BlockSpec((NC, _BLK, F), lambda i: (0, i, 0)),
            pl.BlockSpec((_BLK, 1), lambda i: (i, 0)),
            pl.BlockSpec((_BLK, F), lambda i: (i, 0)),
            pl.BlockSpec((1, F), lambda i: (0, 0)),
            pl.BlockSpec((_BLK, 1), lambda i: (i, 0)),
            pl.BlockSpec((F, F), lambda i: (0, 0)),
            pl.BlockSpec((1, F), lambda i: (0, 0)),
            pl.BlockSpec((F, F), lambda i: (0, 0)),
            pl.BlockSpec((1, F), lambda i: (0, 0)),
        ],
        out_specs=[
            pl.BlockSpec((G, F), lambda i: (0, 0)),
            pl.BlockSpec((G, F), lambda i: (0, 0)),
            pl.BlockSpec((G, 2 * F), lambda i: (0, 0)),
        ],
        out_shape=[
            jax.ShapeDtypeStruct((G, F), jnp.float32),
            jax.ShapeDtypeStruct((G, F), jnp.float32),
            jax.ShapeDtypeStruct((G, 2 * F), jnp.float32),
        ],
    )(agg, cnt, t, b, batch2d, wpT, bp, wgT, bg)


def kernel(x, W1l, b1l, W1r, W2l, b2l, W2r, Wp, bp, Wg, bg, edge_index, batch):
    w1lT, w1rT = W1l.T, W1r.T
    w2lT, w2rT = W2l.T, W2r.T
    wpT = Wp.T
    wgT = (Wg[:, :F] + Wg[:, F:]).T
    b1 = b1l.reshape(1, F)
    b2 = b2l.reshape(1, F)
    bp2 = bp.reshape(1, F)
    bg2 = bg.reshape(1, F)
    batch2d = batch.reshape(N, 1)
    src3 = edge_index[0].reshape(NC * NS, CPW, CHUNK)
    dst3 = edge_index[1].reshape(NC * NS, CPW, CHUNK)

    cnt = _hist(edge_index[1].reshape(E, 1)).reshape(NPAD, 1)
    u1, t1 = _mm2(x, w1lT, w1rT)
    agg1 = _edge_agg(u1, src3, dst3)
    u2, t2 = _combine(agg1, cnt, t1, b1, w2lT, w2rT)
    agg2 = _edge_agg(u2, src3, dst3)
    gv, gate, _ = _pool(agg2, cnt, t2, b2, batch2d, wpT, bp2, wgT, bg2)
    return gv, gate


# speedup vs baseline: 1.0744x; 1.0115x over previous
import functools

import jax
import jax.numpy as jnp
from jax import lax
from jax.experimental import pallas as pl
from jax.experimental.pallas import tpu as pltpu
from jax.experimental.pallas import tpu_sc as plsc

N = 10000
E = 320000
F = 128
G = 64

NC = 2
NS = 16
CHUNK = 40
CHUNKS_PER_W = 250
NPAD = 10240
ROWS_PER_SUB = NPAD // NS
ZROWS = 16

_HIGH = jax.lax.Precision.HIGHEST


CPW = CHUNKS_PER_W
NBUF = 8
DRING = 2 * NBUF
ZROWS2 = 8


def _edge_agg(u, src3, dst3):
    mesh = plsc.VectorSubcoreMesh(core_axis_name="c", subcore_axis_name="s")

    @functools.partial(
        pl.kernel,
        out_type=jax.ShapeDtypeStruct((NC, NPAD, F), jnp.float32),
        mesh=mesh,
        scratch_types=[
            pltpu.VMEM((NBUF, CHUNK), jnp.int32),
            pltpu.VMEM((DRING, CHUNK), jnp.int32),
            pltpu.VMEM((NBUF, CHUNK, F), jnp.float32),
            pltpu.VMEM((ZROWS2, F), jnp.float32),
            pltpu.VMEM_SHARED((NPAD, F), jnp.float32),
            pltpu.SemaphoreType.DMA,
        ] + [pltpu.SemaphoreType.DMA] * (3 * NBUF),
    )
    def k(u_hbm, src_hbm, dst_hbm, agg_hbm,
          src_v, dst_v, rows_v, zf_v, acc_sh, zsem, *sems):
        isem = sems[:NBUF]
        gsem = sems[NBUF:2 * NBUF]
        ssem = sems[2 * NBUF:]
        cid = lax.axis_index("c")
        sid = lax.axis_index("s")
        wid = cid * NS + sid

        for b in range(NBUF):
            pltpu.async_copy(src_hbm.at[wid, b], src_v.at[b], isem[b])
            pltpu.async_copy(dst_hbm.at[wid, b], dst_v.at[b], isem[b])

        @pl.loop(0, ZROWS2)
        def _(i):
            @pl.loop(0, F // 16)
            def _(j):
                zf_v[i, pl.ds(j * 16, 16)] = jnp.zeros((16,), jnp.float32)

        zcopies = [
            pltpu.async_copy(
                zf_v,
                acc_sh.at[pl.ds(sid * ROWS_PER_SUB + z * ZROWS2, ZROWS2)],
                zsem)
            for z in range(ROWS_PER_SUB // ZROWS2)
        ]
        for c in zcopies:
            c.wait()

        plsc.subcore_barrier()

        NITER = (CPW + 2 * NBUF - 1) // (2 * NBUF)

        @pl.loop(0, NITER)
        def _(ii):
            base = ii * 2 * NBUF
            for r in range(2):
                gathers = [None] * NBUF
                for b in range(NBUF):
                    j = base + r * NBUF + b

                    @pl.when(j < CPW)
                    def _(b=b, j=j):
                        pltpu.make_async_copy(
                            src_hbm.at[wid, 0], src_v.at[b], isem[b]).wait()
                        pltpu.make_async_copy(
                            dst_hbm.at[wid, 0], dst_v.at[b % DRING],
                            isem[b]).wait()

                    @pl.when(jnp.logical_and(j < CPW, j >= NBUF))
                    def _(b=b):
                        pltpu.make_async_copy(
                            rows_v.at[b], acc_sh.at[dst_v.at[b]],
                            ssem[b]).wait()

                    @pl.when(j < CPW)
                    def _(b=b):
                        gathers[b] = pltpu.async_copy(
                            u_hbm.at[src_v.at[b]], rows_v.at[b], gsem[b])
                for b in range(NBUF):
                    j = base + r * NBUF + b
                    ds = r * NBUF + b
                    dsn = ((r + 1) % 2) * NBUF + b

                    @pl.when(j < CPW)
                    def _(b=b, ds=ds):
                        pltpu.make_async_copy(
                            u_hbm.at[src_v.at[b]], rows_v.at[b],
                            gsem[b]).wait()
                        pltpu.async_copy(rows_v.at[b], acc_sh.at[dst_v.at[ds]],
                                         ssem[b], add=True)

                    @pl.when(j + NBUF < CPW)
                    def _(b=b, j=j, dsn=dsn):
                        jn = j + NBUF
                        pltpu.async_copy(src_hbm.at[wid, jn], src_v.at[b],
                                         isem[b])
                        pltpu.async_copy(dst_hbm.at[wid, jn], dst_v.at[dsn],
                                        isem[b])

        for b in range(NBUF):
            pltpu.make_async_copy(
                rows_v.at[b], acc_sh.at[dst_v.at[b]], ssem[b]).wait()

        plsc.subcore_barrier()

        rr = sid * ROWS_PER_SUB
        pltpu.sync_copy(acc_sh.at[pl.ds(rr, ROWS_PER_SUB)],
                        agg_hbm.at[cid, pl.ds(rr, ROWS_PER_SUB)])

    return k(u, src3, dst3)


_BLK = 2000


def _mm2_body(x_ref, wl_ref, wr_ref, u_ref, t_ref):
    xb = x_ref[...]
    u_ref[...] = jnp.dot(xb, wl_ref[...], preferred_element_type=jnp.float32,
                         precision=_HIGH)
    t_ref[...] = jnp.dot(xb, wr_ref[...], preferred_element_type=jnp.float32,
                         precision=_HIGH)


def _mm2(x, wlT, wrT):
    return pl.pallas_call(
        _mm2_body,
        grid=(N // _BLK,),
        in_specs=[
            pl.BlockSpec((_BLK, F), lambda i: (i, 0)),
            pl.BlockSpec((F, F), lambda i: (0, 0)),
            pl.BlockSpec((F, F), lambda i: (0, 0)),
        ],
        out_specs=[
            pl.BlockSpec((_BLK, F), lambda i: (i, 0)),
            pl.BlockSpec((_BLK, F), lambda i: (i, 0)),
        ],
        out_shape=[
            jax.ShapeDtypeStruct((N, F), jnp.float32),
            jax.ShapeDtypeStruct((N, F), jnp.float32),
        ],
    )(x, wlT, wrT)


HI = NPAD // F
_HBLK = 4000


def _hist_body(dst_ref, cnt_ref):
    i = pl.program_id(0)
    d = dst_ref[...]
    hi = lax.shift_right_logical(d, 7)
    lo = jnp.bitwise_and(d, 127)
    ohh = (hi == lax.broadcasted_iota(jnp.int32, (_HBLK, HI), 1)
           ).astype(jnp.float32)
    ohl = (lo == lax.broadcasted_iota(jnp.int32, (_HBLK, F), 1)
           ).astype(jnp.float32)
    c = lax.dot_general(ohh, ohl, (((0,), (0,)), ((), ())),
                        preferred_element_type=jnp.float32)

    @pl.when(i == 0)
    def _():
        cnt_ref[...] = c

    @pl.when(i > 0)
    def _():
        cnt_ref[...] += c


def _hist(dst2d):
    return pl.pallas_call(
        _hist_body,
        grid=(E // _HBLK,),
        in_specs=[pl.BlockSpec((_HBLK, 1), lambda i: (i, 0))],
        out_specs=pl.BlockSpec((HI, F), lambda i: (0, 0)),
        out_shape=jax.ShapeDtypeStruct((HI, F), jnp.float32),
    )(dst2d)


def _combine_body(agg_ref, cnt_ref, t_ref, b_ref, wl_ref, wr_ref,
                  u_ref, t2_ref):
    agg = agg_ref[0] + agg_ref[1]
    inv = 1.0 / jnp.maximum(cnt_ref[...], 1.0)
    h = jnp.maximum(agg * inv + t_ref[...] + b_ref[...], 0.0)
    u_ref[...] = jnp.dot(h, wl_ref[...], preferred_element_type=jnp.float32,
                         precision=_HIGH)
    t2_ref[...] = jnp.dot(h, wr_ref[...], preferred_element_type=jnp.float32,
                          precision=_HIGH)


def _combine(agg, cnt, t, b, wlT, wrT):
    return pl.pallas_call(
        _combine_body,
        grid=(N // _BLK,),
        in_specs=[
            pl.BlockSpec((NC, _BLK, F), lambda i: (0, i, 0)),
            pl.BlockSpec((_BLK, 1), lambda i: (i, 0)),
            pl.BlockSpec((_BLK, F), lambda i: (i, 0)),
            pl.BlockSpec((1, F), lambda i: (0, 0)),
            pl.BlockSpec((F, F), lambda i: (0, 0)),
            pl.BlockSpec((F, F), lambda i: (0, 0)),
        ],
        out_specs=[
            pl.BlockSpec((_BLK, F), lambda i: (i, 0)),
            pl.BlockSpec((_BLK, F), lambda i: (i, 0)),
        ],
        out_shape=[
            jax.ShapeDtypeStruct((N, F), jnp.float32),
            jax.ShapeDtypeStruct((N, F), jnp.float32),
        ],
    )(agg, cnt, t, b, wlT, wrT)


def _pool_body(agg_ref, cnt_ref, t_ref, b_ref, batch_ref,
               wp_ref, bp_ref, wg_ref, bg_ref, gv_ref, gate_ref, pe_ref):
    i = pl.program_id(0)
    agg = agg_ref[0] + agg_ref[1]
    inv = 1.0 / jnp.maximum(cnt_ref[...], 1.0)
    h = jnp.maximum(agg * inv + t_ref[...] + b_ref[...], 0.0)
    he = jnp.concatenate([h, jnp.ones_like(h)], axis=1)
    oh = (batch_ref[...] ==
          lax.broadcasted_iota(jnp.int32, (_BLK, G), 1)).astype(jnp.float32)
    pe = lax.dot_general(oh, he, (((0,), (0,)), ((), ())),
                         preferred_element_type=jnp.float32, precision=_HIGH)

    @pl.when(i == 0)
    def _():
        pe_ref[...] = pe

    @pl.when(i > 0)
    def _():
        pe_ref[...] += pe

    @pl.when(i == N // _BLK - 1)
    def _():
        s = pe_ref[:, :F]
        c = pe_ref[:, F:F + 1]
        pooled = s * (1.0 / jnp.maximum(c, 1.0))
        gv = jnp.dot(pooled, wp_ref[...], preferred_element_type=jnp.float32,
                     precision=_HIGH) + bp_ref[...]
        gv_ref[...] = gv
        gate_ref[...] = jnp.dot(gv, wg_ref[...],
                                preferred_element_type=jnp.float32,
                                precision=_HIGH) + bg_ref[...]


def _pool(agg, cnt, t, b, batch2d, wpT, bp, wgT, bg):
    return pl.pallas_call(
        _pool_body,
        grid=(N // _BLK,),
        in_specs=[
            pl.BlockSpec((NC, _BLK, F), lambda i: (0, i, 0)),
            pl.BlockSpec((_BLK, 1), lambda i: (i, 0)),
            pl.BlockSpec((_BLK, F), lambda i: (i, 0)),
            pl.BlockSpec((1, F), lambda i: (0, 0)),
            pl.BlockSpec((_BLK, 1), lambda i: (i, 0)),
            pl.BlockSpec((F, F), lambda i: (0, 0)),
            pl.BlockSpec((1, F), lambda i: (0, 0)),
            pl.BlockSpec((F, F), lambda i: (0, 0)),
            pl.BlockSpec((1, F), lambda i: (0, 0)),
        ],
        out_specs=[
            pl.BlockSpec((G, F), lambda i: (0, 0)),
            pl.BlockSpec((G, F), lambda i: (0, 0)),
            pl.BlockSpec((G, 2 * F), lambda i: (0, 0)),
        ],
        out_shape=[
            jax.ShapeDtypeStruct((G, F), jnp.float32),
            jax.ShapeDtypeStruct((G, F), jnp.float32),
            jax.ShapeDtypeStruct((G, 2 * F), jnp.float32),
        ],
    )(agg, cnt, t, b, batch2d, wpT, bp, wgT, bg)


def kernel(x, W1l, b1l, W1r, W2l, b2l, W2r, Wp, bp, Wg, bg, edge_index, batch):
    w1lT, w1rT = W1l.T, W1r.T
    w2lT, w2rT = W2l.T, W2r.T
    wpT = Wp.T
    wgT = (Wg[:, :F] + Wg[:, F:]).T
    b1 = b1l.reshape(1, F)
    b2 = b2l.reshape(1, F)
    bp2 = bp.reshape(1, F)
    bg2 = bg.reshape(1, F)
    batch2d = batch.reshape(N, 1)
    src3 = edge_index[0].reshape(NC * NS, CPW, CHUNK)
    dst3 = edge_index[1].reshape(NC * NS, CPW, CHUNK)

    cnt = _hist(edge_index[1].reshape(E, 1)).reshape(NPAD, 1)
    u1, t1 = _mm2(x, w1lT, w1rT)
    agg1 = _edge_agg(u1, src3, dst3)
    u2, t2 = _combine(agg1, cnt, t1, b1, w2lT, w2rT)
    agg2 = _edge_agg(u2, src3, dst3)
    gv, gate, _ = _pool(agg2, cnt, t2, b2, batch2d, wpT, bp2, wgT, bg2)
    return gv, gate


# speedup vs baseline: 1.0766x; 1.0021x over previous
import functools

import jax
import jax.numpy as jnp
from jax import lax
from jax.experimental import pallas as pl
from jax.experimental.pallas import tpu as pltpu
from jax.experimental.pallas import tpu_sc as plsc

N = 10000
E = 320000
F = 128
G = 64

NC = 2
NS = 16
CHUNK = 40
CHUNKS_PER_W = 250
NPAD = 10240
ROWS_PER_SUB = NPAD // NS

_HIGH = jax.lax.Precision.HIGHEST


CPW = CHUNKS_PER_W
NBUF = 8
DRING = 2 * NBUF
ZROWS2 = 8


def _edge_agg(u, src3, dst3):
    mesh = plsc.VectorSubcoreMesh(core_axis_name="c", subcore_axis_name="s")

    @functools.partial(
        pl.kernel,
        out_type=jax.ShapeDtypeStruct((NC, NPAD, F), jnp.float32),
        mesh=mesh,
        scratch_types=[
            pltpu.VMEM((NBUF, CHUNK), jnp.int32),
            pltpu.VMEM((DRING, CHUNK), jnp.int32),
            pltpu.VMEM((NBUF, CHUNK, F), jnp.float32),
            pltpu.VMEM((ZROWS2, F), jnp.float32),
            pltpu.VMEM_SHARED((NPAD, F), jnp.float32),
            pltpu.SemaphoreType.DMA,
        ] + [pltpu.SemaphoreType.DMA] * (3 * NBUF),
    )
    def k(u_hbm, src_hbm, dst_hbm, agg_hbm,
          src_v, dst_v, rows_v, zf_v, acc_sh, zsem, *sems):
        isem = sems[:NBUF]
        gsem = sems[NBUF:2 * NBUF]
        ssem = sems[2 * NBUF:]
        cid = lax.axis_index("c")
        sid = lax.axis_index("s")
        wid = cid * NS + sid

        for b in range(NBUF):
            pltpu.async_copy(src_hbm.at[wid, b], src_v.at[b], isem[b])
            pltpu.async_copy(dst_hbm.at[wid, b], dst_v.at[b], isem[b])

        @pl.loop(0, ZROWS2)
        def _(i):
            @pl.loop(0, F // 16)
            def _(j):
                zf_v[i, pl.ds(j * 16, 16)] = jnp.zeros((16,), jnp.float32)

        zcopies = [
            pltpu.async_copy(
                zf_v,
                acc_sh.at[pl.ds(sid * ROWS_PER_SUB + z * ZROWS2, ZROWS2)],
                zsem)
            for z in range(ROWS_PER_SUB // ZROWS2)
        ]
        for c in zcopies:
            c.wait()

        plsc.subcore_barrier()

        NITER = (CPW + 2 * NBUF - 1) // (2 * NBUF)

        @pl.loop(0, NITER)
        def _(ii):
            base = ii * 2 * NBUF
            for r in range(2):
                for b in range(NBUF):
                    j = base + r * NBUF + b

                    @pl.when(j < CPW)
                    def _(b=b, j=j):
                        pltpu.make_async_copy(
                            src_hbm.at[wid, 0], src_v.at[b], isem[b]).wait()
                        pltpu.make_async_copy(
                            dst_hbm.at[wid, 0], dst_v.at[b % DRING],
                            isem[b]).wait()

                    @pl.when(jnp.logical_and(j < CPW, j >= NBUF))
                    def _(b=b):
                        pltpu.make_async_copy(
                            rows_v.at[b], acc_sh.at[dst_v.at[b]],
                            ssem[b]).wait()

                    @pl.when(j < CPW)
                    def _(b=b):
                        pltpu.async_copy(
                            u_hbm.at[src_v.at[b]], rows_v.at[b], gsem[b])
                for b in range(NBUF):
                    j = base + r * NBUF + b
                    ds = r * NBUF + b
                    dsn = ((r + 1) % 2) * NBUF + b

                    @pl.when(j < CPW)
                    def _(b=b, ds=ds):
                        pltpu.make_async_copy(
                            u_hbm.at[src_v.at[b]], rows_v.at[b],
                            gsem[b]).wait()
                        pltpu.async_copy(rows_v.at[b], acc_sh.at[dst_v.at[ds]],
                                         ssem[b], add=True)

                    @pl.when(j + NBUF < CPW)
                    def _(b=b, j=j, dsn=dsn):
                        jn = j + NBUF
                        pltpu.async_copy(src_hbm.at[wid, jn], src_v.at[b],
                                         isem[b])
                        pltpu.async_copy(dst_hbm.at[wid, jn], dst_v.at[dsn],
                                        isem[b])

        for b in range(NBUF):
            pltpu.make_async_copy(
                rows_v.at[b], acc_sh.at[dst_v.at[b]], ssem[b]).wait()

        plsc.subcore_barrier()

        rr = sid * ROWS_PER_SUB
        pltpu.sync_copy(acc_sh.at[pl.ds(rr, ROWS_PER_SUB)],
                        agg_hbm.at[cid, pl.ds(rr, ROWS_PER_SUB)])

    return k(u, src3, dst3)


_BLK = 2000


def _mm2_body(x_ref, wl_ref, wr_ref, u_ref, t_ref):
    xb = x_ref[...]
    u_ref[...] = jnp.dot(xb, wl_ref[...], preferred_element_type=jnp.float32,
                         precision=_HIGH)
    t_ref[...] = jnp.dot(xb, wr_ref[...], preferred_element_type=jnp.float32,
                         precision=_HIGH)


def _mm2(x, wlT, wrT):
    return pl.pallas_call(
        _mm2_body,
        grid=(N // _BLK,),
        in_specs=[
            pl.BlockSpec((_BLK, F), lambda i: (i, 0)),
            pl.BlockSpec((F, F), lambda i: (0, 0)),
            pl.BlockSpec((F, F), lambda i: (0, 0)),
        ],
        out_specs=[
            pl.BlockSpec((_BLK, F), lambda i: (i, 0)),
            pl.BlockSpec((_BLK, F), lambda i: (i, 0)),
        ],
        out_shape=[
            jax.ShapeDtypeStruct((N, F), jnp.float32),
            jax.ShapeDtypeStruct((N, F), jnp.float32),
        ],
    )(x, wlT, wrT)


HI = NPAD // F
_HBLK = 4000


def _hist_body(dst_ref, cnt_ref):
    i = pl.program_id(0)
    d = dst_ref[...]
    hi = lax.shift_right_logical(d, 7)
    lo = jnp.bitwise_and(d, 127)
    ohh = (hi == lax.broadcasted_iota(jnp.int32, (_HBLK, HI), 1)
           ).astype(jnp.float32)
    ohl = (lo == lax.broadcasted_iota(jnp.int32, (_HBLK, F), 1)
           ).astype(jnp.float32)
    c = lax.dot_general(ohh, ohl, (((0,), (0,)), ((), ())),
                        preferred_element_type=jnp.float32)

    @pl.when(i == 0)
    def _():
        cnt_ref[...] = c

    @pl.when(i > 0)
    def _():
        cnt_ref[...] += c


def _hist(dst2d):
    return pl.pallas_call(
        _hist_body,
        grid=(E // _HBLK,),
        in_specs=[pl.BlockSpec((_HBLK, 1), lambda i: (i, 0))],
        out_specs=pl.BlockSpec((HI, F), lambda i: (0, 0)),
        out_shape=jax.ShapeDtypeStruct((HI, F), jnp.float32),
    )(dst2d)


def _combine_body(agg_ref, cnt_ref, t_ref, b_ref, wl_ref, wr_ref,
                  u_ref, t2_ref):
    agg = agg_ref[0] + agg_ref[1]
    inv = 1.0 / jnp.maximum(cnt_ref[...], 1.0)
    h = jnp.maximum(agg * inv + t_ref[...] + b_ref[...], 0.0)
    u_ref[...] = jnp.dot(h, wl_ref[...], preferred_element_type=jnp.float32,
                         precision=_HIGH)
    t2_ref[...] = jnp.dot(h, wr_ref[...], preferred_element_type=jnp.float32,
                          precision=_HIGH)


def _combine(agg, cnt, t, b, wlT, wrT):
    return pl.pallas_call(
        _combine_body,
        grid=(N // _BLK,),
        in_specs=[
            pl.BlockSpec((NC, _BLK, F), lambda i: (0, i, 0)),
            pl.BlockSpec((_BLK, 1), lambda i: (i, 0)),
            pl.BlockSpec((_BLK, F), lambda i: (i, 0)),
            pl.BlockSpec((1, F), lambda i: (0, 0)),
            pl.BlockSpec((F, F), lambda i: (0, 0)),
            pl.BlockSpec((F, F), lambda i: (0, 0)),
        ],
        out_specs=[
            pl.BlockSpec((_BLK, F), lambda i: (i, 0)),
            pl.BlockSpec((_BLK, F), lambda i: (i, 0)),
        ],
        out_shape=[
            jax.ShapeDtypeStruct((N, F), jnp.float32),
            jax.ShapeDtypeStruct((N, F), jnp.float32),
        ],
    )(agg, cnt, t, b, wlT, wrT)


def _pool_body(agg_ref, cnt_ref, t_ref, b_ref, batch_ref,
               wp_ref, bp_ref, wg_ref, bg_ref, gv_ref, gate_ref, pe_ref):
    i = pl.program_id(0)
    agg = agg_ref[0] + agg_ref[1]
    inv = 1.0 / jnp.maximum(cnt_ref[...], 1.0)
    h = jnp.maximum(agg * inv + t_ref[...] + b_ref[...], 0.0)
    he = jnp.concatenate([h, jnp.ones_like(h)], axis=1)
    oh = (batch_ref[...] ==
          lax.broadcasted_iota(jnp.int32, (_BLK, G), 1)).astype(jnp.float32)
    pe = lax.dot_general(oh, he, (((0,), (0,)), ((), ())),
                         preferred_element_type=jnp.float32, precision=_HIGH)

    @pl.when(i == 0)
    def _():
        pe_ref[...] = pe

    @pl.when(i > 0)
    def _():
        pe_ref[...] += pe

    @pl.when(i == N // _BLK - 1)
    def _():
        s = pe_ref[:, :F]
        c = pe_ref[:, F:F + 1]
        pooled = s * (1.0 / jnp.maximum(c, 1.0))
        gv = jnp.dot(pooled, wp_ref[...], preferred_element_type=jnp.float32,
                     precision=_HIGH) + bp_ref[...]
        gv_ref[...] = gv
        gate_ref[...] = jnp.dot(gv, wg_ref[...],
                                preferred_element_type=jnp.float32,
                                precision=_HIGH) + bg_ref[...]


def _pool(agg, cnt, t, b, batch2d, wpT, bp, wgT, bg):
    return pl.pallas_call(
        _pool_body,
        grid=(N // _BLK,),
        in_specs=[
            pl.BlockSpec((NC, _BLK, F), lambda i: (0, i, 0)),
            pl.BlockSpec((_BLK, 1), lambda i: (i, 0)),
            pl.BlockSpec((_BLK, F), lambda i: (i, 0)),
            pl.BlockSpec((1, F), lambda i: (0, 0)),
            pl.BlockSpec((_BLK, 1), lambda i: (i, 0)),
            pl.BlockSpec((F, F), lambda i: (0, 0)),
            pl.BlockSpec((1, F), lambda i: (0, 0)),
            pl.BlockSpec((F, F), lambda i: (0, 0)),
            pl.BlockSpec((1, F), lambda i: (0, 0)),
        ],
        out_specs=[
            pl.BlockSpec((G, F), lambda i: (0, 0)),
            pl.BlockSpec((G, F), lambda i: (0, 0)),
            pl.BlockSpec((G, 2 * F), lambda i: (0, 0)),
        ],
        out_shape=[
            jax.ShapeDtypeStruct((G, F), jnp.float32),
            jax.ShapeDtypeStruct((G, F), jnp.float32),
            jax.ShapeDtypeStruct((G, 2 * F), jnp.float32),
        ],
    )(agg, cnt, t, b, batch2d, wpT, bp, wgT, bg)


def kernel(x, W1l, b1l, W1r, W2l, b2l, W2r, Wp, bp, Wg, bg, edge_index, batch):
    w1lT, w1rT = W1l.T, W1r.T
    w2lT, w2rT = W2l.T, W2r.T
    wpT = Wp.T
    wgT = (Wg[:, :F] + Wg[:, F:]).T
    b1 = b1l.reshape(1, F)
    b2 = b2l.reshape(1, F)
    bp2 = bp.reshape(1, F)
    bg2 = bg.reshape(1, F)
    batch2d = batch.reshape(N, 1)
    src3 = edge_index[0].reshape(NC * NS, CPW, CHUNK)
    dst3 = edge_index[1].reshape(NC * NS, CPW, CHUNK)

    cnt = _hist(edge_index[1].reshape(E, 1)).reshape(NPAD, 1)
    u1, t1 = _mm2(x, w1lT, w1rT)
    agg1 = _edge_agg(u1, src3, dst3)
    u2, t2 = _combine(agg1, cnt, t1, b1, w2lT, w2rT)
    agg2 = _edge_agg(u2, src3, dst3)
    gv, gate, _ = _pool(agg2, cnt, t2, b2, batch2d, wpT, bp2, wgT, bg2)
    return gv, gate
